# Initial kernel scaffold; baseline (speedup 1.0000x reference)
import jax
import jax.numpy as jnp
from jax.experimental import pallas as pl


def kernel(x, edge_index, batch, W1, b1, W2, b2, W3, b3, Wl, bl):
    raise NotImplementedError("write your pallas kernel here")


# speedup vs baseline: 26.9361x; 26.9361x over previous
import functools

import jax
import jax.numpy as jnp
from jax import lax
from jax.experimental import pallas as pl
from jax.experimental.pallas import tpu as pltpu
from jax.experimental.pallas import tpu_sc as plsc

N = 10000
NP = 10240
E = 320000
F = 128
H = 32

NC, NS = 2, 16
NW = NC * NS
EPW = E // NW
CS = 80
CH = EPW // CS
RPT = NP // NS

_MESH = plsc.VectorSubcoreMesh(core_axis_name="c", subcore_axis_name="s")
_SC_PARAMS = pltpu.CompilerParams(use_tc_tiling_on_sc=False)


def _zero16():
    return jnp.zeros((16,), jnp.float32)


@functools.partial(
    pl.kernel,
    out_type=jax.ShapeDtypeStruct((NC, 1, NP), jnp.float32),
    mesh=_MESH,
    compiler_params=_SC_PARAMS,
    scratch_types=[
        pltpu.VMEM((CH, CS), jnp.int32),
        pltpu.VMEM((CS,), jnp.float32),
        pltpu.VMEM((RPT,), jnp.float32),
        pltpu.VMEM_SHARED((NP,), jnp.float32),
    ],
)
def _deg_kernel(dst_hbm, degp_hbm, idx_v, ones_v, z_v, acc_sh):
    c = lax.axis_index("c")
    s = lax.axis_index("s")
    w = s * NC + c

    for i in range(CS // 16):
        ones_v[pl.ds(16 * i, 16)] = jnp.ones((16,), jnp.float32)
    for i in range(RPT // 16):
        z_v[pl.ds(16 * i, 16)] = _zero16()

    pltpu.sync_copy(dst_hbm.at[w], idx_v)
    pltpu.sync_copy(z_v, acc_sh.at[pl.ds(s * RPT, RPT)])
    plsc.subcore_barrier()

    for j in range(CH):
        pltpu.sync_copy(ones_v, acc_sh.at[idx_v.at[j]], add=True)

    plsc.subcore_barrier()
    pltpu.sync_copy(acc_sh.at[pl.ds(s * RPT, RPT)],
                    degp_hbm.at[c, 0, pl.ds(s * RPT, RPT)])


@functools.partial(
    pl.kernel,
    out_type=jax.ShapeDtypeStruct((NC, NP, H), jnp.float32),
    mesh=_MESH,
    compiler_params=_SC_PARAMS,
    scratch_types=[
        pltpu.VMEM((CH, CS), jnp.int32),
        pltpu.VMEM((CH, CS), jnp.int32),
        pltpu.VMEM((CS, H), jnp.float32),
        pltpu.VMEM((CS, H), jnp.float32),
        pltpu.VMEM((CS, H), jnp.float32),
        pltpu.VMEM_SHARED((NP, H), jnp.float32),
        pltpu.SemaphoreType.DMA,
        pltpu.SemaphoreType.DMA,
    ],
)
def _agg_kernel(hh_hbm, src_hbm, dst_hbm, out_hbm,
                sidx, didx, b0, b1, zrow, acc_sh, gsem, ssem):
    c = lax.axis_index("c")
    s = lax.axis_index("s")
    w = s * NC + c
    bufs = (b0, b1)

    for r in range(CS):
        zrow[r, pl.ds(0, 16)] = _zero16()
        zrow[r, pl.ds(16, 16)] = _zero16()

    pltpu.sync_copy(src_hbm.at[w], sidx)
    pltpu.sync_copy(dst_hbm.at[w], didx)
    for t in range(RPT // CS):
        pltpu.sync_copy(zrow, acc_sh.at[pl.ds(s * RPT + t * CS, CS)])
    plsc.subcore_barrier()

    gd = [None] * CH
    sd = [None] * CH
    gd[0] = pltpu.async_copy(hh_hbm.at[sidx.at[0]], bufs[0], gsem)
    for j in range(CH):
        gd[j].wait()
        sd[j] = pltpu.async_copy(bufs[j % 2], acc_sh.at[didx.at[j]], ssem,
                                 add=True)
        if j + 1 < CH:
            if j >= 1:
                sd[j - 1].wait()
            gd[j + 1] = pltpu.async_copy(hh_hbm.at[sidx.at[j + 1]],
                                         bufs[(j + 1) % 2], gsem)
    sd[CH - 2].wait()
    sd[CH - 1].wait()

    plsc.subcore_barrier()
    pltpu.sync_copy(acc_sh.at[pl.ds(s * RPT, RPT)],
                    out_hbm.at[c, pl.ds(s * RPT, RPT)])


BLK = 1024
GRID = NP // BLK


def _tc1_body(degp_ref, x_ref, w_ref, dinv_ref, hh_ref):
    deg = degp_ref[0] + degp_ref[1] + 1.0
    dinv = lax.rsqrt(deg)
    dinvc = dinv.reshape(BLK, 1)
    h = jnp.dot(x_ref[...], w_ref[...], preferred_element_type=jnp.float32)
    dinv_ref[...] = dinvc
    hh_ref[...] = h * dinvc


def _tc_first(degp, x, w1):
    return pl.pallas_call(
        _tc1_body,
        grid=(GRID,),
        in_specs=[
            pl.BlockSpec((NC, 1, BLK), lambda i: (0, 0, i)),
            pl.BlockSpec((BLK, F), lambda i: (i, 0)),
            pl.BlockSpec((F, H), lambda i: (0, 0)),
        ],
        out_specs=[
            pl.BlockSpec((BLK, 1), lambda i: (i, 0)),
            pl.BlockSpec((BLK, H), lambda i: (i, 0)),
        ],
        out_shape=[
            jax.ShapeDtypeStruct((NP, 1), jnp.float32),
            jax.ShapeDtypeStruct((NP, H), jnp.float32),
        ],
    )(degp, x, w1)


def _tc_mid_body(p_ref, hh_ref, dinv_ref, b_ref, w_ref, hho_ref):
    dinvc = dinv_ref[...]
    t = dinvc * (p_ref[0] + p_ref[1] + hh_ref[...]) + b_ref[...]
    a = jax.nn.sigmoid(t)
    hho_ref[...] = dinvc * jnp.dot(a, w_ref[...],
                                   preferred_element_type=jnp.float32)


def _tc_mid(p, hh, dinv, b, w):
    return pl.pallas_call(
        _tc_mid_body,
        grid=(GRID,),
        in_specs=[
            pl.BlockSpec((NC, BLK, H), lambda i: (0, i, 0)),
            pl.BlockSpec((BLK, H), lambda i: (i, 0)),
            pl.BlockSpec((BLK, 1), lambda i: (i, 0)),
            pl.BlockSpec((1, H), lambda i: (0, 0)),
            pl.BlockSpec((H, H), lambda i: (0, 0)),
        ],
        out_specs=pl.BlockSpec((BLK, H), lambda i: (i, 0)),
        out_shape=jax.ShapeDtypeStruct((NP, H), jnp.float32),
    )(p, hh, dinv, b, w)


def _tc_fin_body(p_ref, hh_ref, dinv_ref, b3_ref, wl_ref, bl_ref, out_ref):
    dinvc = dinv_ref[...]
    t = dinvc * (p_ref[0] + p_ref[1] + hh_ref[...]) + b3_ref[...]
    out_ref[...] = jax.nn.sigmoid(
        jnp.dot(t, wl_ref[...], preferred_element_type=jnp.float32)
        + bl_ref[...])


def _tc_fin(p, hh, dinv, b3, wl, bl):
    return pl.pallas_call(
        _tc_fin_body,
        grid=(GRID,),
        in_specs=[
            pl.BlockSpec((NC, BLK, H), lambda i: (0, i, 0)),
            pl.BlockSpec((BLK, H), lambda i: (i, 0)),
            pl.BlockSpec((BLK, 1), lambda i: (i, 0)),
            pl.BlockSpec((1, H), lambda i: (0, 0)),
            pl.BlockSpec((H, F), lambda i: (0, 0)),
            pl.BlockSpec((1, F), lambda i: (0, 0)),
        ],
        out_specs=pl.BlockSpec((BLK, F), lambda i: (i, 0)),
        out_shape=jax.ShapeDtypeStruct((NP, F), jnp.float32),
    )(p, hh, dinv, b3, wl, bl)


def kernel(x, edge_index, batch, W1, b1, W2, b2, W3, b3, Wl, bl):
    del batch
    src = edge_index[0].reshape(NW, CH, CS)
    dst = edge_index[1].reshape(NW, CH, CS)

    xp = jnp.pad(x, ((0, NP - N), (0, 0)))

    degp = _deg_kernel(dst)
    dinv, hh = _tc_first(degp, xp, W1)

    p1 = _agg_kernel(hh, src, dst)
    hh2 = _tc_mid(p1, hh, dinv, b1.reshape(1, H), W2)

    p2 = _agg_kernel(hh2, src, dst)
    hh3 = _tc_mid(p2, hh2, dinv, b2.reshape(1, H), W3)

    p3 = _agg_kernel(hh3, src, dst)
    out = _tc_fin(p3, hh3, dinv, b3.reshape(1, H), Wl, bl.reshape(1, F))

    return out[:N]


# speedup vs baseline: 46.8938x; 1.7409x over previous
import functools

import jax
import jax.numpy as jnp
from jax import lax
from jax.experimental import pallas as pl
from jax.experimental.pallas import tpu as pltpu
from jax.experimental.pallas import tpu_sc as plsc

N = 10000
NP = 10240
E = 320000
F = 128
H = 32

NC, NS = 2, 16
NW = NC * NS
EPW = E // NW
CS = 80
CH = EPW // CS
RPT = NP // NS

_MESH = plsc.VectorSubcoreMesh(core_axis_name="c", subcore_axis_name="s")
_SC_PARAMS = pltpu.CompilerParams(use_tc_tiling_on_sc=False)


def _zero16():
    return jnp.zeros((16,), jnp.float32)


@functools.partial(
    pl.kernel,
    out_type=jax.ShapeDtypeStruct((NC, 1, NP), jnp.float32),
    mesh=_MESH,
    compiler_params=_SC_PARAMS,
    scratch_types=[
        pltpu.VMEM((CH, CS), jnp.int32),
        pltpu.VMEM((CS,), jnp.float32),
        pltpu.VMEM((RPT,), jnp.float32),
        pltpu.VMEM_SHARED((NP,), jnp.float32),
    ],
)
def _deg_kernel(dst_hbm, degp_hbm, idx_v, ones_v, z_v, acc_sh):
    c = lax.axis_index("c")
    s = lax.axis_index("s")
    w = s * NC + c

    for i in range(CS // 16):
        ones_v[pl.ds(16 * i, 16)] = jnp.ones((16,), jnp.float32)
    for i in range(RPT // 16):
        z_v[pl.ds(16 * i, 16)] = _zero16()

    pltpu.sync_copy(dst_hbm.at[w], idx_v)
    pltpu.sync_copy(z_v, acc_sh.at[pl.ds(s * RPT, RPT)])
    plsc.subcore_barrier()

    for j in range(CH):
        pltpu.sync_copy(ones_v, acc_sh.at[idx_v.at[j]], add=True)

    plsc.subcore_barrier()
    pltpu.sync_copy(acc_sh.at[pl.ds(s * RPT, RPT)],
                    degp_hbm.at[c, 0, pl.ds(s * RPT, RPT)])


@functools.partial(
    pl.kernel,
    out_type=jax.ShapeDtypeStruct((NC, NP, H), jnp.float32),
    mesh=_MESH,
    compiler_params=_SC_PARAMS,
    scratch_types=[
        pltpu.VMEM((CH, CS), jnp.int32),
        pltpu.VMEM((CH, CS), jnp.int32),
        pltpu.VMEM((8, CS, H), jnp.float32),
        pltpu.VMEM((CS, H), jnp.float32),
        pltpu.VMEM_SHARED((NP, H), jnp.float32),
        pltpu.SemaphoreType.DMA,
        pltpu.SemaphoreType.DMA,
    ],
)
def _agg_kernel(hh_hbm, src_hbm, dst_hbm, out_hbm,
                sidx, didx, ring, zrow, acc_sh, gsem, ssem):
    c = lax.axis_index("c")
    s = lax.axis_index("s")
    w = s * NC + c
    D = 4
    NBUF = 2 * D

    for r in range(CS):
        zrow[r, pl.ds(0, 16)] = _zero16()
        zrow[r, pl.ds(16, 16)] = _zero16()

    pltpu.sync_copy(src_hbm.at[w], sidx)
    pltpu.sync_copy(dst_hbm.at[w], didx)
    for t in range(RPT // CS):
        pltpu.sync_copy(zrow, acc_sh.at[pl.ds(s * RPT + t * CS, CS)])
    plsc.subcore_barrier()

    gd = [None] * CH
    sd = [None] * CH
    for j in range(D):
        gd[j] = pltpu.async_copy(hh_hbm.at[sidx.at[j]], ring.at[j % NBUF],
                                 gsem)
    for j in range(CH):
        gd[j].wait()
        sd[j] = pltpu.async_copy(ring.at[j % NBUF], acc_sh.at[didx.at[j]],
                                 ssem, add=True)
        if j + D < CH:
            if j >= D:
                sd[j - D].wait()
            gd[j + D] = pltpu.async_copy(hh_hbm.at[sidx.at[j + D]],
                                         ring.at[(j + D) % NBUF], gsem)
    for j in range(max(0, CH - 2 * D), CH):
        sd[j].wait()

    plsc.subcore_barrier()
    pltpu.sync_copy(acc_sh.at[pl.ds(s * RPT, RPT)],
                    out_hbm.at[c, pl.ds(s * RPT, RPT)])


BLK = 1024
GRID = NP // BLK


def _tc1_body(degp_ref, x_ref, w_ref, dinv_ref, hh_ref):
    deg = degp_ref[0] + degp_ref[1] + 1.0
    dinv = lax.rsqrt(deg)
    dinvc = dinv.reshape(BLK, 1)
    h = jnp.dot(x_ref[...], w_ref[...], preferred_element_type=jnp.float32)
    dinv_ref[...] = dinvc
    hh_ref[...] = h * dinvc


def _tc_first(degp, x, w1):
    return pl.pallas_call(
        _tc1_body,
        grid=(GRID,),
        in_specs=[
            pl.BlockSpec((NC, 1, BLK), lambda i: (0, 0, i)),
            pl.BlockSpec((BLK, F), lambda i: (i, 0)),
            pl.BlockSpec((F, H), lambda i: (0, 0)),
        ],
        out_specs=[
            pl.BlockSpec((BLK, 1), lambda i: (i, 0)),
            pl.BlockSpec((BLK, H), lambda i: (i, 0)),
        ],
        out_shape=[
            jax.ShapeDtypeStruct((NP, 1), jnp.float32),
            jax.ShapeDtypeStruct((NP, H), jnp.float32),
        ],
    )(degp, x, w1)


def _tc_mid_body(p_ref, hh_ref, dinv_ref, b_ref, w_ref, hho_ref):
    dinvc = dinv_ref[...]
    t = dinvc * (p_ref[0] + p_ref[1] + hh_ref[...]) + b_ref[...]
    a = jax.nn.sigmoid(t)
    hho_ref[...] = dinvc * jnp.dot(a, w_ref[...],
                                   preferred_element_type=jnp.float32)


def _tc_mid(p, hh, dinv, b, w):
    return pl.pallas_call(
        _tc_mid_body,
        grid=(GRID,),
        in_specs=[
            pl.BlockSpec((NC, BLK, H), lambda i: (0, i, 0)),
            pl.BlockSpec((BLK, H), lambda i: (i, 0)),
            pl.BlockSpec((BLK, 1), lambda i: (i, 0)),
            pl.BlockSpec((1, H), lambda i: (0, 0)),
            pl.BlockSpec((H, H), lambda i: (0, 0)),
        ],
        out_specs=pl.BlockSpec((BLK, H), lambda i: (i, 0)),
        out_shape=jax.ShapeDtypeStruct((NP, H), jnp.float32),
    )(p, hh, dinv, b, w)


def _tc_fin_body(p_ref, hh_ref, dinv_ref, b3_ref, wl_ref, bl_ref, out_ref):
    dinvc = dinv_ref[...]
    t = dinvc * (p_ref[0] + p_ref[1] + hh_ref[...]) + b3_ref[...]
    out_ref[...] = jax.nn.sigmoid(
        jnp.dot(t, wl_ref[...], preferred_element_type=jnp.float32)
        + bl_ref[...])


def _tc_fin(p, hh, dinv, b3, wl, bl):
    return pl.pallas_call(
        _tc_fin_body,
        grid=(GRID,),
        in_specs=[
            pl.BlockSpec((NC, BLK, H), lambda i: (0, i, 0)),
            pl.BlockSpec((BLK, H), lambda i: (i, 0)),
            pl.BlockSpec((BLK, 1), lambda i: (i, 0)),
            pl.BlockSpec((1, H), lambda i: (0, 0)),
            pl.BlockSpec((H, F), lambda i: (0, 0)),
            pl.BlockSpec((1, F), lambda i: (0, 0)),
        ],
        out_specs=pl.BlockSpec((BLK, F), lambda i: (i, 0)),
        out_shape=jax.ShapeDtypeStruct((NP, F), jnp.float32),
    )(p, hh, dinv, b3, wl, bl)


def kernel(x, edge_index, batch, W1, b1, W2, b2, W3, b3, Wl, bl):
    del batch
    src = edge_index[0].reshape(NW, CH, CS)
    dst = edge_index[1].reshape(NW, CH, CS)

    xp = jnp.pad(x, ((0, NP - N), (0, 0)))

    degp = _deg_kernel(dst)
    dinv, hh = _tc_first(degp, xp, W1)

    p1 = _agg_kernel(hh, src, dst)
    hh2 = _tc_mid(p1, hh, dinv, b1.reshape(1, H), W2)

    p2 = _agg_kernel(hh2, src, dst)
    hh3 = _tc_mid(p2, hh2, dinv, b2.reshape(1, H), W3)

    p3 = _agg_kernel(hh3, src, dst)
    out = _tc_fin(p3, hh3, dinv, b3.reshape(1, H), Wl, bl.reshape(1, F))

    return out[:N]


# speedup vs baseline: 49.0932x; 1.0469x over previous
import functools

import jax
import jax.numpy as jnp
from jax import lax
from jax.experimental import pallas as pl
from jax.experimental.pallas import tpu as pltpu
from jax.experimental.pallas import tpu_sc as plsc

N = 10000
NP = 10240
E = 320000
F = 128
H = 32

NC, NS = 2, 16
NW = NC * NS
EPW = E // NW
CS = 80
CH = EPW // CS
RPT = NP // NS

_MESH = plsc.VectorSubcoreMesh(core_axis_name="c", subcore_axis_name="s")
_SC_PARAMS = pltpu.CompilerParams(use_tc_tiling_on_sc=False)


def _zero16():
    return jnp.zeros((16,), jnp.float32)


@functools.partial(
    pl.kernel,
    out_type=jax.ShapeDtypeStruct((NC, 1, NP), jnp.float32),
    mesh=_MESH,
    compiler_params=_SC_PARAMS,
    scratch_types=[
        pltpu.VMEM((CH, CS), jnp.int32),
        pltpu.VMEM((CS,), jnp.float32),
        pltpu.VMEM((RPT,), jnp.float32),
        pltpu.VMEM_SHARED((NP,), jnp.float32),
    ],
)
def _deg_kernel(edge_hbm, degp_hbm, idx_v, ones_v, z_v, acc_sh):
    c = lax.axis_index("c")
    s = lax.axis_index("s")
    w = s * NC + c

    for i in range(CS // 16):
        ones_v[pl.ds(16 * i, 16)] = jnp.ones((16,), jnp.float32)
    for i in range(RPT // 16):
        z_v[pl.ds(16 * i, 16)] = _zero16()

    pltpu.sync_copy(edge_hbm.at[1, w], idx_v)
    pltpu.sync_copy(z_v, acc_sh.at[pl.ds(s * RPT, RPT)])
    plsc.subcore_barrier()

    for j in range(CH):
        pltpu.sync_copy(ones_v, acc_sh.at[idx_v.at[j]], add=True)

    plsc.subcore_barrier()
    pltpu.sync_copy(acc_sh.at[pl.ds(s * RPT, RPT)],
                    degp_hbm.at[c, 0, pl.ds(s * RPT, RPT)])


@functools.partial(
    pl.kernel,
    out_type=jax.ShapeDtypeStruct((NC, NP, H), jnp.float32),
    mesh=_MESH,
    compiler_params=_SC_PARAMS,
    scratch_types=[
        pltpu.VMEM((CH, CS), jnp.int32),
        pltpu.VMEM((CH, CS), jnp.int32),
        pltpu.VMEM((8, CS, H), jnp.float32),
        pltpu.VMEM((CS, H), jnp.float32),
        pltpu.VMEM_SHARED((NP, H), jnp.float32),
        pltpu.SemaphoreType.DMA,
        pltpu.SemaphoreType.DMA,
    ],
)
def _agg_kernel(hh_hbm, edge_hbm, out_hbm,
                sidx, didx, ring, zrow, acc_sh, gsem, ssem):
    c = lax.axis_index("c")
    s = lax.axis_index("s")
    w = s * NC + c
    D = 4
    NBUF = 2 * D

    for r in range(CS):
        zrow[r, pl.ds(0, 16)] = _zero16()
        zrow[r, pl.ds(16, 16)] = _zero16()

    pltpu.sync_copy(edge_hbm.at[0, w], sidx)
    pltpu.sync_copy(edge_hbm.at[1, w], didx)
    for t in range(RPT // CS):
        pltpu.sync_copy(zrow, acc_sh.at[pl.ds(s * RPT + t * CS, CS)])
    plsc.subcore_barrier()

    gd = [None] * CH
    sd = [None] * CH
    for j in range(D):
        gd[j] = pltpu.async_copy(hh_hbm.at[sidx.at[j]], ring.at[j % NBUF],
                                 gsem)
    for j in range(CH):
        gd[j].wait()
        sd[j] = pltpu.async_copy(ring.at[j % NBUF], acc_sh.at[didx.at[j]],
                                 ssem, add=True)
        if j + D < CH:
            if j >= D:
                sd[j - D].wait()
            gd[j + D] = pltpu.async_copy(hh_hbm.at[sidx.at[j + D]],
                                         ring.at[(j + D) % NBUF], gsem)
    for j in range(max(0, CH - 2 * D), CH):
        sd[j].wait()

    plsc.subcore_barrier()
    pltpu.sync_copy(acc_sh.at[pl.ds(s * RPT, RPT)],
                    out_hbm.at[c, pl.ds(s * RPT, RPT)])


BLK = 1024
GRID = NP // BLK


def _tc1_body(degp_ref, x_ref, w_ref, dinv_ref, hh_ref):
    deg = degp_ref[0] + degp_ref[1] + 1.0
    dinv = lax.rsqrt(deg)
    dinvc = dinv.reshape(BLK, 1)
    h = jnp.dot(x_ref[...], w_ref[...], preferred_element_type=jnp.float32)
    dinv_ref[...] = dinvc
    hh_ref[...] = h * dinvc


def _tc_first(degp, x, w1):
    return pl.pallas_call(
        _tc1_body,
        grid=(GRID,),
        in_specs=[
            pl.BlockSpec((NC, 1, BLK), lambda i: (0, 0, i)),
            pl.BlockSpec((BLK, F), lambda i: (i, 0)),
            pl.BlockSpec((F, H), lambda i: (0, 0)),
        ],
        out_specs=[
            pl.BlockSpec((BLK, 1), lambda i: (i, 0)),
            pl.BlockSpec((BLK, H), lambda i: (i, 0)),
        ],
        out_shape=[
            jax.ShapeDtypeStruct((NP, 1), jnp.float32),
            jax.ShapeDtypeStruct((NP, H), jnp.float32),
        ],
    )(degp, x, w1)


def _tc_mid_body(p_ref, hh_ref, dinv_ref, b_ref, w_ref, hho_ref):
    dinvc = dinv_ref[...]
    t = dinvc * (p_ref[0] + p_ref[1] + hh_ref[...]) + b_ref[...]
    a = jax.nn.sigmoid(t)
    hho_ref[...] = dinvc * jnp.dot(a, w_ref[...],
                                   preferred_element_type=jnp.float32)


def _tc_mid(p, hh, dinv, b, w):
    return pl.pallas_call(
        _tc_mid_body,
        grid=(GRID,),
        in_specs=[
            pl.BlockSpec((NC, BLK, H), lambda i: (0, i, 0)),
            pl.BlockSpec((BLK, H), lambda i: (i, 0)),
            pl.BlockSpec((BLK, 1), lambda i: (i, 0)),
            pl.BlockSpec((1, H), lambda i: (0, 0)),
            pl.BlockSpec((H, H), lambda i: (0, 0)),
        ],
        out_specs=pl.BlockSpec((BLK, H), lambda i: (i, 0)),
        out_shape=jax.ShapeDtypeStruct((NP, H), jnp.float32),
    )(p, hh, dinv, b, w)


def _tc_fin_body(p_ref, hh_ref, dinv_ref, b3_ref, wl_ref, bl_ref, out_ref):
    dinvc = dinv_ref[...]
    t = dinvc * (p_ref[0] + p_ref[1] + hh_ref[...]) + b3_ref[...]
    out_ref[...] = jax.nn.sigmoid(
        jnp.dot(t, wl_ref[...], preferred_element_type=jnp.float32)
        + bl_ref[...])


def _tc_fin(p, hh, dinv, b3, wl, bl):
    return pl.pallas_call(
        _tc_fin_body,
        grid=(GRID,),
        in_specs=[
            pl.BlockSpec((NC, BLK, H), lambda i: (0, i, 0)),
            pl.BlockSpec((BLK, H), lambda i: (i, 0)),
            pl.BlockSpec((BLK, 1), lambda i: (i, 0)),
            pl.BlockSpec((1, H), lambda i: (0, 0)),
            pl.BlockSpec((H, F), lambda i: (0, 0)),
            pl.BlockSpec((1, F), lambda i: (0, 0)),
        ],
        out_specs=pl.BlockSpec((BLK, F), lambda i: (i, 0)),
        out_shape=jax.ShapeDtypeStruct((NP, F), jnp.float32),
    )(p, hh, dinv, b3, wl, bl)


def kernel(x, edge_index, batch, W1, b1, W2, b2, W3, b3, Wl, bl):
    del batch
    e4 = edge_index.reshape(2, NW, CH, CS)

    xp = jnp.pad(x, ((0, NP - N), (0, 0)))

    degp = _deg_kernel(e4)
    dinv, hh = _tc_first(degp, xp, W1)

    p1 = _agg_kernel(hh, e4)
    hh2 = _tc_mid(p1, hh, dinv, b1.reshape(1, H), W2)

    p2 = _agg_kernel(hh2, e4)
    hh3 = _tc_mid(p2, hh2, dinv, b2.reshape(1, H), W3)

    p3 = _agg_kernel(hh3, e4)
    out = _tc_fin(p3, hh3, dinv, b3.reshape(1, H), Wl, bl.reshape(1, F))

    return out[:N]


# speedup vs baseline: 51.7254x; 1.0536x over previous
import functools

import jax
import jax.numpy as jnp
from jax import lax
from jax.experimental import pallas as pl
from jax.experimental.pallas import tpu as pltpu
from jax.experimental.pallas import tpu_sc as plsc

N = 10000
NP = 10240
E = 320000
F = 128
H = 32

NC, NS = 2, 16
NW = NC * NS
EPW = E // NW
CS = 80
CH = EPW // CS
RPT = NP // NS

_MESH = plsc.VectorSubcoreMesh(core_axis_name="c", subcore_axis_name="s")
_SC_PARAMS = pltpu.CompilerParams(use_tc_tiling_on_sc=False)


def _zero16():
    return jnp.zeros((16,), jnp.float32)


@functools.partial(
    pl.kernel,
    out_type=jax.ShapeDtypeStruct((NC, 1, NP), jnp.float32),
    mesh=_MESH,
    compiler_params=_SC_PARAMS,
    scratch_types=[
        pltpu.VMEM((CH, CS), jnp.int32),
        pltpu.VMEM((CS,), jnp.float32),
        pltpu.VMEM((RPT,), jnp.float32),
        pltpu.VMEM_SHARED((NP,), jnp.float32),
    ],
)
def _deg_kernel(edge_hbm, degp_hbm, idx_v, ones_v, z_v, acc_sh):
    c = lax.axis_index("c")
    s = lax.axis_index("s")
    w = s * NC + c

    for i in range(CS // 16):
        ones_v[pl.ds(16 * i, 16)] = jnp.ones((16,), jnp.float32)
    for i in range(RPT // 16):
        z_v[pl.ds(16 * i, 16)] = _zero16()

    pltpu.sync_copy(edge_hbm.at[1, w], idx_v)
    pltpu.sync_copy(z_v, acc_sh.at[pl.ds(s * RPT, RPT)])
    plsc.subcore_barrier()

    for j in range(CH):
        pltpu.sync_copy(ones_v, acc_sh.at[idx_v.at[j]], add=True)

    plsc.subcore_barrier()
    pltpu.sync_copy(acc_sh.at[pl.ds(s * RPT, RPT)],
                    degp_hbm.at[c, 0, pl.ds(s * RPT, RPT)])


@functools.partial(
    pl.kernel,
    out_type=jax.ShapeDtypeStruct((NC, NP, H), jnp.float32),
    mesh=_MESH,
    compiler_params=_SC_PARAMS,
    scratch_types=[
        pltpu.VMEM((CH, CS), jnp.int32),
        pltpu.VMEM((CH, CS), jnp.int32),
        pltpu.VMEM((8, CS, H), jnp.float32),
        pltpu.VMEM((CS, H), jnp.float32),
        pltpu.VMEM_SHARED((NP, H), jnp.float32),
        pltpu.SemaphoreType.DMA,
        pltpu.SemaphoreType.DMA,
    ],
)
def _agg_kernel(hh_hbm, edge_hbm, out_hbm,
                sidx, didx, ring, zrow, acc_sh, gsem, ssem):
    c = lax.axis_index("c")
    s = lax.axis_index("s")
    w = s * NC + c
    D = 4
    NBUF = 2 * D

    for r in range(CS):
        zrow[r, pl.ds(0, 16)] = _zero16()
        zrow[r, pl.ds(16, 16)] = _zero16()

    pltpu.sync_copy(edge_hbm.at[0, w], sidx)
    pltpu.sync_copy(edge_hbm.at[1, w], didx)
    for t in range(RPT // CS):
        pltpu.sync_copy(zrow, acc_sh.at[pl.ds(s * RPT + t * CS, CS)])
    plsc.subcore_barrier()

    gd = [None] * CH
    sd = [None] * CH
    for j in range(D):
        gd[j] = pltpu.async_copy(hh_hbm.at[sidx.at[j]], ring.at[j % NBUF],
                                 gsem)
    for j in range(CH):
        gd[j].wait()
        sd[j] = pltpu.async_copy(ring.at[j % NBUF], acc_sh.at[didx.at[j]],
                                 ssem, add=True)
        if j + D < CH:
            if j >= D:
                sd[j - D].wait()
            gd[j + D] = pltpu.async_copy(hh_hbm.at[sidx.at[j + D]],
                                         ring.at[(j + D) % NBUF], gsem)
    for j in range(max(0, CH - 2 * D), CH):
        sd[j].wait()

    plsc.subcore_barrier()
    pltpu.sync_copy(acc_sh.at[pl.ds(s * RPT, RPT)],
                    out_hbm.at[c, pl.ds(s * RPT, RPT)])


BLK = 2048
GRID = NP // BLK


def _dinv_col(degp_ref):
    deg = degp_ref[0] + degp_ref[1] + 1.0
    return lax.rsqrt(deg).reshape(BLK, 1)


def _tc1_body(degp_ref, x_ref, w_ref, hh_ref):
    h = jnp.dot(x_ref[...], w_ref[...], preferred_element_type=jnp.float32)
    hh_ref[...] = h * _dinv_col(degp_ref)


def _tc_first(degp, x, w1):
    return pl.pallas_call(
        _tc1_body,
        grid=(GRID,),
        in_specs=[
            pl.BlockSpec((NC, 1, BLK), lambda i: (0, 0, i)),
            pl.BlockSpec((BLK, F), lambda i: (i, 0)),
            pl.BlockSpec((F, H), lambda i: (0, 0)),
        ],
        out_specs=pl.BlockSpec((BLK, H), lambda i: (i, 0)),
        out_shape=jax.ShapeDtypeStruct((NP, H), jnp.float32),
    )(degp, x, w1)


def _tc_mid_body(degp_ref, p_ref, hh_ref, b_ref, w_ref, hho_ref):
    dinvc = _dinv_col(degp_ref)
    t = dinvc * (p_ref[0] + p_ref[1] + hh_ref[...]) + b_ref[...]
    a = jax.nn.sigmoid(t)
    hho_ref[...] = dinvc * jnp.dot(a, w_ref[...],
                                   preferred_element_type=jnp.float32)


def _tc_mid(degp, p, hh, b, w):
    return pl.pallas_call(
        _tc_mid_body,
        grid=(GRID,),
        in_specs=[
            pl.BlockSpec((NC, 1, BLK), lambda i: (0, 0, i)),
            pl.BlockSpec((NC, BLK, H), lambda i: (0, i, 0)),
            pl.BlockSpec((BLK, H), lambda i: (i, 0)),
            pl.BlockSpec((1, H), lambda i: (0, 0)),
            pl.BlockSpec((H, H), lambda i: (0, 0)),
        ],
        out_specs=pl.BlockSpec((BLK, H), lambda i: (i, 0)),
        out_shape=jax.ShapeDtypeStruct((NP, H), jnp.float32),
    )(degp, p, hh, b, w)


def _tc_fin_body(degp_ref, p_ref, hh_ref, b3_ref, wl_ref, bl_ref, out_ref):
    dinvc = _dinv_col(degp_ref)
    t = dinvc * (p_ref[0] + p_ref[1] + hh_ref[...]) + b3_ref[...]
    out_ref[...] = jax.nn.sigmoid(
        jnp.dot(t, wl_ref[...], preferred_element_type=jnp.float32)
        + bl_ref[...])


def _tc_fin(degp, p, hh, b3, wl, bl):
    return pl.pallas_call(
        _tc_fin_body,
        grid=(GRID,),
        in_specs=[
            pl.BlockSpec((NC, 1, BLK), lambda i: (0, 0, i)),
            pl.BlockSpec((NC, BLK, H), lambda i: (0, i, 0)),
            pl.BlockSpec((BLK, H), lambda i: (i, 0)),
            pl.BlockSpec((1, H), lambda i: (0, 0)),
            pl.BlockSpec((H, F), lambda i: (0, 0)),
            pl.BlockSpec((1, F), lambda i: (0, 0)),
        ],
        out_specs=pl.BlockSpec((BLK, F), lambda i: (i, 0)),
        out_shape=jax.ShapeDtypeStruct((NP, F), jnp.float32),
    )(degp, p, hh, b3, wl, bl)


def kernel(x, edge_index, batch, W1, b1, W2, b2, W3, b3, Wl, bl):
    del batch
    e4 = edge_index.reshape(2, NW, CH, CS)

    xp = jnp.pad(x, ((0, NP - N), (0, 0)))

    degp = _deg_kernel(e4)
    hh = _tc_first(degp, xp, W1)

    p1 = _agg_kernel(hh, e4)
    hh2 = _tc_mid(degp, p1, hh, b1.reshape(1, H), W2)

    p2 = _agg_kernel(hh2, e4)
    hh3 = _tc_mid(degp, p2, hh2, b2.reshape(1, H), W3)

    p3 = _agg_kernel(hh3, e4)
    out = _tc_fin(degp, p3, hh3, b3.reshape(1, H), Wl, bl.reshape(1, F))

    return out[:N]


# speedup vs baseline: 53.5310x; 1.0349x over previous
import functools

import jax
import jax.numpy as jnp
from jax import lax
from jax.experimental import pallas as pl
from jax.experimental.pallas import tpu as pltpu
from jax.experimental.pallas import tpu_sc as plsc

N = 10000
NP = 10240
E = 320000
F = 128
H = 32

NC, NS = 2, 16
NW = NC * NS
EPW = E // NW
CS = 80
CH = EPW // CS
RPT = NP // NS

_MESH = plsc.VectorSubcoreMesh(core_axis_name="c", subcore_axis_name="s")
_SC_PARAMS = pltpu.CompilerParams(use_tc_tiling_on_sc=False)


def _zero16():
    return jnp.zeros((16,), jnp.float32)


@functools.partial(
    pl.kernel,
    out_type=jax.ShapeDtypeStruct((NC, 1, NP), jnp.float32),
    mesh=_MESH,
    compiler_params=_SC_PARAMS,
    scratch_types=[
        pltpu.VMEM((CH, CS), jnp.int32),
        pltpu.VMEM((CS,), jnp.float32),
        pltpu.VMEM((RPT,), jnp.float32),
        pltpu.VMEM_SHARED((NP,), jnp.float32),
        pltpu.SemaphoreType.DMA,
    ],
)
def _deg_kernel(edge_hbm, degp_hbm, idx_v, ones_v, z_v, acc_sh, ssem):
    c = lax.axis_index("c")
    s = lax.axis_index("s")
    w = s * NC + c

    for i in range(CS // 16):
        ones_v[pl.ds(16 * i, 16)] = jnp.ones((16,), jnp.float32)
    for i in range(RPT // 16):
        z_v[pl.ds(16 * i, 16)] = _zero16()

    pltpu.sync_copy(edge_hbm.at[1, w], idx_v)
    pltpu.sync_copy(z_v, acc_sh.at[pl.ds(s * RPT, RPT)])
    plsc.subcore_barrier()

    sd = [pltpu.async_copy(ones_v, acc_sh.at[idx_v.at[j]], ssem, add=True)
          for j in range(CH)]
    for d in sd:
        d.wait()

    plsc.subcore_barrier()
    pltpu.sync_copy(acc_sh.at[pl.ds(s * RPT, RPT)],
                    degp_hbm.at[c, 0, pl.ds(s * RPT, RPT)])


@functools.partial(
    pl.kernel,
    out_type=jax.ShapeDtypeStruct((NC, NP, H), jnp.float32),
    mesh=_MESH,
    compiler_params=_SC_PARAMS,
    scratch_types=[
        pltpu.VMEM((CH, CS), jnp.int32),
        pltpu.VMEM((CH, CS), jnp.int32),
        pltpu.VMEM((8, CS, H), jnp.float32),
        pltpu.VMEM((CS, H), jnp.float32),
        pltpu.VMEM_SHARED((NP, H), jnp.float32),
        pltpu.SemaphoreType.DMA,
        pltpu.SemaphoreType.DMA,
    ],
)
def _agg_kernel(hh_hbm, edge_hbm, out_hbm,
                sidx, didx, ring, zrow, acc_sh, gsem, ssem):
    c = lax.axis_index("c")
    s = lax.axis_index("s")
    w = s * NC + c
    D = 4
    NBUF = 2 * D

    for r in range(CS):
        zrow[r, pl.ds(0, 16)] = _zero16()
        zrow[r, pl.ds(16, 16)] = _zero16()

    pltpu.sync_copy(edge_hbm.at[0, w], sidx)
    pltpu.sync_copy(edge_hbm.at[1, w], didx)
    for t in range(RPT // CS):
        pltpu.sync_copy(zrow, acc_sh.at[pl.ds(s * RPT + t * CS, CS)])
    plsc.subcore_barrier()

    gd = [None] * CH
    sd = [None] * CH
    for j in range(D):
        gd[j] = pltpu.async_copy(hh_hbm.at[sidx.at[j]], ring.at[j % NBUF],
                                 gsem)
    for j in range(CH):
        gd[j].wait()
        sd[j] = pltpu.async_copy(ring.at[j % NBUF], acc_sh.at[didx.at[j]],
                                 ssem, add=True)
        if j + D < CH:
            if j >= D:
                sd[j - D].wait()
            gd[j + D] = pltpu.async_copy(hh_hbm.at[sidx.at[j + D]],
                                         ring.at[(j + D) % NBUF], gsem)
    for j in range(max(0, CH - 2 * D), CH):
        sd[j].wait()

    plsc.subcore_barrier()
    pltpu.sync_copy(acc_sh.at[pl.ds(s * RPT, RPT)],
                    out_hbm.at[c, pl.ds(s * RPT, RPT)])


BLK = 2048
GRID = NP // BLK


def _dinv_col(degp_ref):
    deg = degp_ref[0] + degp_ref[1] + 1.0
    return lax.rsqrt(deg).reshape(BLK, 1)


def _tc1_body(degp_ref, x_ref, w_ref, hh_ref):
    h = jnp.dot(x_ref[...], w_ref[...], preferred_element_type=jnp.float32)
    hh_ref[...] = h * _dinv_col(degp_ref)


def _tc_first(degp, x, w1):
    return pl.pallas_call(
        _tc1_body,
        grid=(GRID,),
        in_specs=[
            pl.BlockSpec((NC, 1, BLK), lambda i: (0, 0, i)),
            pl.BlockSpec((BLK, F), lambda i: (i, 0)),
            pl.BlockSpec((F, H), lambda i: (0, 0)),
        ],
        out_specs=pl.BlockSpec((BLK, H), lambda i: (i, 0)),
        out_shape=jax.ShapeDtypeStruct((NP, H), jnp.float32),
    )(degp, x, w1)


def _tc_mid_body(degp_ref, p_ref, hh_ref, b_ref, w_ref, hho_ref):
    dinvc = _dinv_col(degp_ref)
    t = dinvc * (p_ref[0] + p_ref[1] + hh_ref[...]) + b_ref[...]
    a = jax.nn.sigmoid(t)
    hho_ref[...] = dinvc * jnp.dot(a, w_ref[...],
                                   preferred_element_type=jnp.float32)


def _tc_mid(degp, p, hh, b, w):
    return pl.pallas_call(
        _tc_mid_body,
        grid=(GRID,),
        in_specs=[
            pl.BlockSpec((NC, 1, BLK), lambda i: (0, 0, i)),
            pl.BlockSpec((NC, BLK, H), lambda i: (0, i, 0)),
            pl.BlockSpec((BLK, H), lambda i: (i, 0)),
            pl.BlockSpec((1, H), lambda i: (0, 0)),
            pl.BlockSpec((H, H), lambda i: (0, 0)),
        ],
        out_specs=pl.BlockSpec((BLK, H), lambda i: (i, 0)),
        out_shape=jax.ShapeDtypeStruct((NP, H), jnp.float32),
    )(degp, p, hh, b, w)


def _tc_fin_body(degp_ref, p_ref, hh_ref, b3_ref, wl_ref, bl_ref, out_ref):
    dinvc = _dinv_col(degp_ref)
    t = dinvc * (p_ref[0] + p_ref[1] + hh_ref[...]) + b3_ref[...]
    out_ref[...] = jax.nn.sigmoid(
        jnp.dot(t, wl_ref[...], preferred_element_type=jnp.float32)
        + bl_ref[...])


def _tc_fin(degp, p, hh, b3, wl, bl):
    return pl.pallas_call(
        _tc_fin_body,
        grid=(GRID,),
        in_specs=[
            pl.BlockSpec((NC, 1, BLK), lambda i: (0, 0, i)),
            pl.BlockSpec((NC, BLK, H), lambda i: (0, i, 0)),
            pl.BlockSpec((BLK, H), lambda i: (i, 0)),
            pl.BlockSpec((1, H), lambda i: (0, 0)),
            pl.BlockSpec((H, F), lambda i: (0, 0)),
            pl.BlockSpec((1, F), lambda i: (0, 0)),
        ],
        out_specs=pl.BlockSpec((BLK, F), lambda i: (i, 0)),
        out_shape=jax.ShapeDtypeStruct((NP, F), jnp.float32),
    )(degp, p, hh, b3, wl, bl)


def kernel(x, edge_index, batch, W1, b1, W2, b2, W3, b3, Wl, bl):
    del batch
    e4 = edge_index.reshape(2, NW, CH, CS)

    xp = jnp.pad(x, ((0, NP - N), (0, 0)))

    degp = _deg_kernel(e4)
    hh = _tc_first(degp, xp, W1)

    p1 = _agg_kernel(hh, e4)
    hh2 = _tc_mid(degp, p1, hh, b1.reshape(1, H), W2)

    p2 = _agg_kernel(hh2, e4)
    hh3 = _tc_mid(degp, p2, hh2, b2.reshape(1, H), W3)

    p3 = _agg_kernel(hh3, e4)
    out = _tc_fin(degp, p3, hh3, b3.reshape(1, H), Wl, bl.reshape(1, F))

    return out[:N]


# speedup vs baseline: 66.6884x; 1.2458x over previous
import functools

import jax
import jax.numpy as jnp
from jax import lax
from jax.experimental import pallas as pl
from jax.experimental.pallas import tpu as pltpu
from jax.experimental.pallas import tpu_sc as plsc

N = 10000
NP = 10240
E = 320000
F = 128
H = 32
NP4 = NP // 4

NC, NS = 2, 16
NW = NC * NS
EPW = E // NW
CS = 80
CH = EPW // CS
RPT = NP // NS

_MESH = plsc.VectorSubcoreMesh(core_axis_name="c", subcore_axis_name="s")
_SC_PARAMS = pltpu.CompilerParams(use_tc_tiling_on_sc=False)
_SC_PARAMS_NL = pltpu.CompilerParams(use_tc_tiling_on_sc=False,
                                     needs_layout_passes=False)


def _zero16():
    return jnp.zeros((16,), jnp.float32)


@functools.partial(
    pl.kernel,
    out_type=jax.ShapeDtypeStruct((NC, NP * H), jnp.float32),
    mesh=_MESH,
    compiler_params=_SC_PARAMS_NL,
    scratch_types=[
        pltpu.VMEM((CH, CS), jnp.int32),
        pltpu.VMEM((CS,), jnp.float32),
        pltpu.VMEM((RPT,), jnp.float32),
        pltpu.VMEM((RPT * H,), jnp.float32),
        pltpu.VMEM_SHARED((NP,), jnp.float32),
        pltpu.SemaphoreType.DMA,
    ],
)
def _deg_kernel(edge_hbm, degx_hbm, idx_v, ones_v, z_v, dexp_v, acc_sh, ssem):
    c = lax.axis_index("c")
    s = lax.axis_index("s")
    w = s * NC + c

    for i in range(CS // 16):
        ones_v[pl.ds(16 * i, 16)] = jnp.ones((16,), jnp.float32)
    for i in range(RPT // 16):
        z_v[pl.ds(16 * i, 16)] = _zero16()

    pltpu.sync_copy(edge_hbm.at[1, w], idx_v)
    pltpu.sync_copy(z_v, acc_sh.at[pl.ds(s * RPT, RPT)])
    plsc.subcore_barrier()

    sd = [pltpu.async_copy(ones_v, acc_sh.at[idx_v.at[j]], ssem, add=True)
          for j in range(CH)]
    for d in sd:
        d.wait()

    plsc.subcore_barrier()
    pltpu.sync_copy(acc_sh.at[pl.ds(s * RPT, RPT)], z_v)

    def _expand(v, carry):
        row = plsc.load_gather(z_v, [jnp.full((16,), v >> 1, jnp.int32)])
        dexp_v[pl.ds(16 * v, 16)] = row
        return carry

    lax.fori_loop(0, RPT * H // 16, _expand, 0)
    pltpu.sync_copy(dexp_v, degx_hbm.at[c, pl.ds(s * RPT * H, RPT * H)])


@functools.partial(
    pl.kernel,
    out_type=jax.ShapeDtypeStruct((NC, NP, H), jnp.float32),
    mesh=_MESH,
    compiler_params=_SC_PARAMS,
    scratch_types=[
        pltpu.VMEM((CH, CS), jnp.int32),
        pltpu.VMEM((CH, CS), jnp.int32),
        pltpu.VMEM((8, CS, H), jnp.float32),
        pltpu.VMEM((CS, H), jnp.float32),
        pltpu.VMEM_SHARED((NP, H), jnp.float32),
        pltpu.SemaphoreType.DMA,
        pltpu.SemaphoreType.DMA,
    ],
)
def _agg_kernel(hh_hbm, edge_hbm, out_hbm,
                sidx, didx, ring, zrow, acc_sh, gsem, ssem):
    c = lax.axis_index("c")
    s = lax.axis_index("s")
    w = s * NC + c
    D = 4
    NBUF = 2 * D

    for r in range(CS):
        zrow[r, pl.ds(0, 16)] = _zero16()
        zrow[r, pl.ds(16, 16)] = _zero16()

    pltpu.sync_copy(edge_hbm.at[0, w], sidx)
    pltpu.sync_copy(edge_hbm.at[1, w], didx)
    for t in range(RPT // CS):
        pltpu.sync_copy(zrow, acc_sh.at[pl.ds(s * RPT + t * CS, CS)])
    plsc.subcore_barrier()

    gd = [None] * CH
    sd = [None] * CH
    for j in range(D):
        gd[j] = pltpu.async_copy(hh_hbm.at[sidx.at[j]], ring.at[j % NBUF],
                                 gsem)
    for j in range(CH):
        gd[j].wait()
        sd[j] = pltpu.async_copy(ring.at[j % NBUF], acc_sh.at[didx.at[j]],
                                 ssem, add=True)
        if j + D < CH:
            if j >= D:
                sd[j - D].wait()
            gd[j + D] = pltpu.async_copy(hh_hbm.at[sidx.at[j + D]],
                                         ring.at[(j + D) % NBUF], gsem)
    for j in range(max(0, CH - 2 * D), CH):
        sd[j].wait()

    plsc.subcore_barrier()
    pltpu.sync_copy(acc_sh.at[pl.ds(s * RPT, RPT)],
                    out_hbm.at[c, pl.ds(s * RPT, RPT)])


BLK = 512
GRID = NP4 // BLK


def _dinvp(dx_ref):
    return lax.rsqrt(dx_ref[0] + dx_ref[1] + 1.0)


def _tc1_body(dx_ref, x_ref, w_ref, hh_ref):
    h = jnp.dot(x_ref[...], w_ref[...], preferred_element_type=jnp.float32)
    hh_ref[...] = h * _dinvp(dx_ref)


def _tc_first(degx4, x4, w1k):
    return pl.pallas_call(
        _tc1_body,
        grid=(GRID,),
        in_specs=[
            pl.BlockSpec((NC, BLK, F), lambda i: (0, i, 0)),
            pl.BlockSpec((BLK, 4 * F), lambda i: (i, 0)),
            pl.BlockSpec((4 * F, F), lambda i: (0, 0)),
        ],
        out_specs=pl.BlockSpec((BLK, F), lambda i: (i, 0)),
        out_shape=jax.ShapeDtypeStruct((NP4, F), jnp.float32),
    )(degx4, x4, w1k)


def _tc_mid_body(dx_ref, p_ref, hh_ref, b_ref, w_ref, hho_ref):
    dinvp = _dinvp(dx_ref)
    t = dinvp * (p_ref[0] + p_ref[1] + hh_ref[...]) + b_ref[...]
    a = jax.nn.sigmoid(t)
    hho_ref[...] = dinvp * jnp.dot(a, w_ref[...],
                                   preferred_element_type=jnp.float32)


def _tc_mid(degx4, p4, hhp, b4, wk):
    return pl.pallas_call(
        _tc_mid_body,
        grid=(GRID,),
        in_specs=[
            pl.BlockSpec((NC, BLK, F), lambda i: (0, i, 0)),
            pl.BlockSpec((NC, BLK, F), lambda i: (0, i, 0)),
            pl.BlockSpec((BLK, F), lambda i: (i, 0)),
            pl.BlockSpec((1, F), lambda i: (0, 0)),
            pl.BlockSpec((F, F), lambda i: (0, 0)),
        ],
        out_specs=pl.BlockSpec((BLK, F), lambda i: (i, 0)),
        out_shape=jax.ShapeDtypeStruct((NP4, F), jnp.float32),
    )(degx4, p4, hhp, b4, wk)


def _tc_fin_body(dx_ref, p_ref, hh_ref, b3_ref, wl_ref, bl_ref, out_ref):
    t = _dinvp(dx_ref) * (p_ref[0] + p_ref[1] + hh_ref[...]) + b3_ref[...]
    out_ref[...] = jax.nn.sigmoid(
        jnp.dot(t, wl_ref[...], preferred_element_type=jnp.float32)
        + bl_ref[...])


def _tc_fin(degx4, p4, hhp, b34, wlk, bl4):
    return pl.pallas_call(
        _tc_fin_body,
        grid=(GRID,),
        in_specs=[
            pl.BlockSpec((NC, BLK, F), lambda i: (0, i, 0)),
            pl.BlockSpec((NC, BLK, F), lambda i: (0, i, 0)),
            pl.BlockSpec((BLK, F), lambda i: (i, 0)),
            pl.BlockSpec((1, F), lambda i: (0, 0)),
            pl.BlockSpec((F, 4 * F), lambda i: (0, 0)),
            pl.BlockSpec((1, 4 * F), lambda i: (0, 0)),
        ],
        out_specs=pl.BlockSpec((BLK, 4 * F), lambda i: (i, 0)),
        out_shape=jax.ShapeDtypeStruct((NP4, 4 * F), jnp.float32),
    )(degx4, p4, hhp, b34, wlk, bl4)


def kernel(x, edge_index, batch, W1, b1, W2, b2, W3, b3, Wl, bl):
    del batch
    e4 = edge_index.reshape(2, NW, CH, CS)
    eye4 = jnp.eye(4, dtype=jnp.float32)
    w1k = jnp.kron(eye4, W1)
    w2k = jnp.kron(eye4, W2)
    w3k = jnp.kron(eye4, W3)
    wlk = jnp.kron(eye4, Wl)
    b14 = jnp.tile(b1, 4).reshape(1, F)
    b24 = jnp.tile(b2, 4).reshape(1, F)
    b34 = jnp.tile(b3, 4).reshape(1, F)
    bl4 = jnp.tile(bl, 4).reshape(1, 4 * F)

    x4 = jnp.pad(x, ((0, NP - N), (0, 0))).reshape(NP4, 4 * F)

    degx = _deg_kernel(e4)
    degx4 = degx.reshape(NC, NP4, F)

    hh1p = _tc_first(degx4, x4, w1k)

    p1 = _agg_kernel(hh1p.reshape(NP, H), e4)
    hh2p = _tc_mid(degx4, p1.reshape(NC, NP4, F), hh1p, b14, w2k)

    p2 = _agg_kernel(hh2p.reshape(NP, H), e4)
    hh3p = _tc_mid(degx4, p2.reshape(NC, NP4, F), hh2p, b24, w3k)

    p3 = _agg_kernel(hh3p.reshape(NP, H), e4)
    out4 = _tc_fin(degx4, p3.reshape(NC, NP4, F), hh3p, b34, wlk, bl4)

    return out4.reshape(NP, F)[:N]


# speedup vs baseline: 69.4213x; 1.0410x over previous
import functools

import jax
import jax.numpy as jnp
from jax import lax
from jax.experimental import pallas as pl
from jax.experimental.pallas import tpu as pltpu
from jax.experimental.pallas import tpu_sc as plsc

N = 10000
NP = 10240
E = 320000
F = 128
H = 32
NP4 = NP // 4

NC, NS = 2, 16
NW = NC * NS
EPW = E // NW
CS = 80
CH = EPW // CS
RPT = NP // NS

_MESH = plsc.VectorSubcoreMesh(core_axis_name="c", subcore_axis_name="s")
_SC_PARAMS = pltpu.CompilerParams(use_tc_tiling_on_sc=False)
_SC_PARAMS_NL = pltpu.CompilerParams(use_tc_tiling_on_sc=False,
                                     needs_layout_passes=False)


def _zero16():
    return jnp.zeros((16,), jnp.float32)


@functools.partial(
    pl.kernel,
    out_type=jax.ShapeDtypeStruct((NC, NP * H), jnp.float32),
    mesh=_MESH,
    compiler_params=_SC_PARAMS_NL,
    scratch_types=[
        pltpu.VMEM((CH, CS), jnp.int32),
        pltpu.VMEM((CS,), jnp.float32),
        pltpu.VMEM((RPT,), jnp.float32),
        pltpu.VMEM((RPT * H,), jnp.float32),
        pltpu.VMEM_SHARED((NP,), jnp.float32),
        pltpu.SemaphoreType.DMA,
    ],
)
def _deg_kernel(edge_hbm, degx_hbm, idx_v, ones_v, z_v, dexp_v, acc_sh, ssem):
    c = lax.axis_index("c")
    s = lax.axis_index("s")
    w = s * NC + c

    for i in range(CS // 16):
        ones_v[pl.ds(16 * i, 16)] = jnp.ones((16,), jnp.float32)
    for i in range(RPT // 16):
        z_v[pl.ds(16 * i, 16)] = _zero16()

    pltpu.sync_copy(edge_hbm.at[1, w], idx_v)
    pltpu.sync_copy(z_v, acc_sh.at[pl.ds(s * RPT, RPT)])
    plsc.subcore_barrier()

    sd = [pltpu.async_copy(ones_v, acc_sh.at[idx_v.at[j]], ssem, add=True)
          for j in range(CH)]
    for d in sd:
        d.wait()

    plsc.subcore_barrier()
    pltpu.sync_copy(acc_sh.at[pl.ds(s * RPT, RPT)], z_v)

    def _expand(v, carry):
        row = plsc.load_gather(z_v, [jnp.full((16,), v >> 1, jnp.int32)])
        dexp_v[pl.ds(16 * v, 16)] = row
        return carry

    lax.fori_loop(0, RPT * H // 16, _expand, 0)
    pltpu.sync_copy(dexp_v, degx_hbm.at[c, pl.ds(s * RPT * H, RPT * H)])


@functools.partial(
    pl.kernel,
    out_type=jax.ShapeDtypeStruct((NC, NP, H), jnp.float32),
    mesh=_MESH,
    compiler_params=_SC_PARAMS,
    scratch_types=[
        pltpu.VMEM((CH, CS), jnp.int32),
        pltpu.VMEM((CH, CS), jnp.int32),
        pltpu.VMEM((8, CS, H), jnp.float32),
        pltpu.VMEM((CS, H), jnp.float32),
        pltpu.VMEM_SHARED((NP, H), jnp.float32),
        pltpu.SemaphoreType.DMA,
        pltpu.SemaphoreType.DMA,
        pltpu.SemaphoreType.DMA,
    ],
)
def _agg_kernel(hh_hbm, edge_hbm, out_hbm,
                sidx, didx, ring, zrow, acc_sh, gsem, ssem, zsem):
    c = lax.axis_index("c")
    s = lax.axis_index("s")
    w = s * NC + c
    D = 4
    NBUF = 2 * D

    sld = pltpu.async_copy(edge_hbm.at[0, w], sidx, gsem)
    dld = pltpu.async_copy(edge_hbm.at[1, w], didx, ssem)
    for r in range(CS):
        zrow[r, pl.ds(0, 16)] = _zero16()
        zrow[r, pl.ds(16, 16)] = _zero16()
    zd = [pltpu.async_copy(zrow, acc_sh.at[pl.ds(s * RPT + t * CS, CS)], zsem)
          for t in range(RPT // CS)]

    gd = [None] * CH
    sd = [None] * CH
    sld.wait()
    for j in range(D):
        gd[j] = pltpu.async_copy(hh_hbm.at[sidx.at[j]], ring.at[j % NBUF],
                                 gsem)
    dld.wait()
    for d in zd:
        d.wait()
    plsc.subcore_barrier()

    for j in range(CH):
        gd[j].wait()
        sd[j] = pltpu.async_copy(ring.at[j % NBUF], acc_sh.at[didx.at[j]],
                                 ssem, add=True)
        if j + D < CH:
            if j >= D:
                sd[j - D].wait()
            gd[j + D] = pltpu.async_copy(hh_hbm.at[sidx.at[j + D]],
                                         ring.at[(j + D) % NBUF], gsem)
    for j in range(max(0, CH - 2 * D), CH):
        sd[j].wait()

    plsc.subcore_barrier()
    pltpu.sync_copy(acc_sh.at[pl.ds(s * RPT, RPT)],
                    out_hbm.at[c, pl.ds(s * RPT, RPT)])


BLK = 512
GRID = NP4 // BLK


def _dinvp(dx_ref):
    return lax.rsqrt(dx_ref[0] + dx_ref[1] + 1.0)


def _tc1_body(dx_ref, x_ref, w_ref, hh_ref):
    h = jnp.dot(x_ref[...], w_ref[...], preferred_element_type=jnp.float32)
    hh_ref[...] = h * _dinvp(dx_ref)


def _tc_first(degx4, x4, w1k):
    return pl.pallas_call(
        _tc1_body,
        grid=(GRID,),
        in_specs=[
            pl.BlockSpec((NC, BLK, F), lambda i: (0, i, 0)),
            pl.BlockSpec((BLK, 4 * F), lambda i: (i, 0)),
            pl.BlockSpec((4 * F, F), lambda i: (0, 0)),
        ],
        out_specs=pl.BlockSpec((BLK, F), lambda i: (i, 0)),
        out_shape=jax.ShapeDtypeStruct((NP4, F), jnp.float32),
    )(degx4, x4, w1k)


def _tc_mid_body(dx_ref, p_ref, hh_ref, b_ref, w_ref, hho_ref):
    dinvp = _dinvp(dx_ref)
    t = dinvp * (p_ref[0] + p_ref[1] + hh_ref[...]) + b_ref[...]
    a = jax.nn.sigmoid(t)
    hho_ref[...] = dinvp * jnp.dot(a, w_ref[...],
                                   preferred_element_type=jnp.float32)


def _tc_mid(degx4, p4, hhp, b4, wk):
    return pl.pallas_call(
        _tc_mid_body,
        grid=(GRID,),
        in_specs=[
            pl.BlockSpec((NC, BLK, F), lambda i: (0, i, 0)),
            pl.BlockSpec((NC, BLK, F), lambda i: (0, i, 0)),
            pl.BlockSpec((BLK, F), lambda i: (i, 0)),
            pl.BlockSpec((1, F), lambda i: (0, 0)),
            pl.BlockSpec((F, F), lambda i: (0, 0)),
        ],
        out_specs=pl.BlockSpec((BLK, F), lambda i: (i, 0)),
        out_shape=jax.ShapeDtypeStruct((NP4, F), jnp.float32),
    )(degx4, p4, hhp, b4, wk)


def _tc_fin_body(dx_ref, p_ref, hh_ref, b3_ref, wl_ref, bl_ref, out_ref):
    t = _dinvp(dx_ref) * (p_ref[0] + p_ref[1] + hh_ref[...]) + b3_ref[...]
    out_ref[...] = jax.nn.sigmoid(
        jnp.dot(t, wl_ref[...], preferred_element_type=jnp.float32)
        + bl_ref[...])


def _tc_fin(degx4, p4, hhp, b34, wlk, bl4):
    return pl.pallas_call(
        _tc_fin_body,
        grid=(GRID,),
        in_specs=[
            pl.BlockSpec((NC, BLK, F), lambda i: (0, i, 0)),
            pl.BlockSpec((NC, BLK, F), lambda i: (0, i, 0)),
            pl.BlockSpec((BLK, F), lambda i: (i, 0)),
            pl.BlockSpec((1, F), lambda i: (0, 0)),
            pl.BlockSpec((F, 4 * F), lambda i: (0, 0)),
            pl.BlockSpec((1, 4 * F), lambda i: (0, 0)),
        ],
        out_specs=pl.BlockSpec((BLK, 4 * F), lambda i: (i, 0)),
        out_shape=jax.ShapeDtypeStruct((NP4, 4 * F), jnp.float32),
    )(degx4, p4, hhp, b34, wlk, bl4)


def kernel(x, edge_index, batch, W1, b1, W2, b2, W3, b3, Wl, bl):
    del batch
    e4 = edge_index.reshape(2, NW, CH, CS)
    eye4 = jnp.eye(4, dtype=jnp.float32)
    w1k = jnp.kron(eye4, W1)
    w2k = jnp.kron(eye4, W2)
    w3k = jnp.kron(eye4, W3)
    wlk = jnp.kron(eye4, Wl)
    b14 = jnp.tile(b1, 4).reshape(1, F)
    b24 = jnp.tile(b2, 4).reshape(1, F)
    b34 = jnp.tile(b3, 4).reshape(1, F)
    bl4 = jnp.tile(bl, 4).reshape(1, 4 * F)

    x4 = jnp.pad(x, ((0, NP - N), (0, 0))).reshape(NP4, 4 * F)

    degx = _deg_kernel(e4)
    degx4 = degx.reshape(NC, NP4, F)

    hh1p = _tc_first(degx4, x4, w1k)

    p1 = _agg_kernel(hh1p.reshape(NP, H), e4)
    hh2p = _tc_mid(degx4, p1.reshape(NC, NP4, F), hh1p, b14, w2k)

    p2 = _agg_kernel(hh2p.reshape(NP, H), e4)
    hh3p = _tc_mid(degx4, p2.reshape(NC, NP4, F), hh2p, b24, w3k)

    p3 = _agg_kernel(hh3p.reshape(NP, H), e4)
    out4 = _tc_fin(degx4, p3.reshape(NC, NP4, F), hh3p, b34, wlk, bl4)

    return out4.reshape(NP, F)[:N]
